# baseline (speedup 1.0000x reference)
import jax
import jax.numpy as jnp
from jax.experimental import pallas as pl
from jax.experimental.pallas import tpu as pltpu

TEMP = 0.1
N = 4096
D = 256
BM = 1024
GRID = N // BM
EXP2_SCALE = 1.0 / (TEMP * 0.6931471805599453)
SQRT_SCALE = EXP2_SCALE ** 0.5


def _top_candidates(arr, n_top):
    cand = []
    cur = arr
    zero = jnp.array(0.0, arr.dtype)
    for it in range(n_top):
        m = jnp.max(cur, axis=1, keepdims=True)
        cand.append(m)
        if it + 1 < n_top:
            cur = jnp.where(cur == m, zero, cur)
    return cand


def _sort4(v1, v2, v3, v4):
    s1, t1 = jnp.maximum(v1, v2), jnp.minimum(v1, v2)
    s2, t2 = jnp.maximum(v3, v4), jnp.minimum(v3, v4)
    l1, x = jnp.maximum(s1, s2), jnp.minimum(s1, s2)
    y, l4 = jnp.maximum(t1, t2), jnp.minimum(t1, t2)
    l2, l3 = jnp.maximum(x, y), jnp.minimum(x, y)
    return l1, l2, l3, l4


def _quarters(arr):
    q = arr.shape[1] // 4
    return arr[:, 0:q], arr[:, q:2 * q], arr[:, 2 * q:3 * q], arr[:, 3 * q:]


def _halves(arr):
    h = arr.shape[1] // 2
    a, b = arr[:, 0:h], arr[:, h:]
    return jnp.maximum(a, b), jnp.minimum(a, b)


def _and_kernel(z_ref, out_ref, zn_ref):
    i = pl.program_id(0)

    @pl.when(i == 0)
    def _init():
        zf = z_ref[...]
        nrm = jnp.sqrt(jnp.sum(zf * zf, axis=1, keepdims=True))
        zn = zf / jnp.maximum(nrm, 1e-12)
        zn_ref[...] = (zn * SQRT_SCALE).astype(jnp.bfloat16)
        out_ref[...] = jnp.zeros_like(out_ref)

    zn_blk = zn_ref[pl.ds(i * BM, BM), :]
    sim = jax.lax.dot_general(
        zn_blk,
        zn_ref[...],
        dimension_numbers=(((1,), (1,)), ((), ())),
        preferred_element_type=jnp.float32,
    )

    m0 = jnp.max(sim, axis=1, keepdims=True)
    e = jnp.exp2(sim - m0)

    eb = e.astype(jnp.bfloat16)
    ones = jnp.ones((N, 1), jnp.bfloat16)
    den = jax.lax.dot_general(
        eb,
        ones,
        dimension_numbers=(((1,), (0,)), ((), ())),
        preferred_element_type=jnp.float32,
    ) - 1.0
    l1, l2, l3, l4 = _sort4(*_quarters(eb))
    b1, b2, b3, b4 = _sort4(*_quarters(l1))

    cand = []
    one = jnp.array(1.0, b1.dtype)
    zero = jnp.array(0.0, b1.dtype)
    cand += _top_candidates(jnp.where(b1 == one, zero, b1), 5)
    cand += _top_candidates(b2, 3)
    cand += _top_candidates(b3, 2)
    cand.append(jnp.max(b4, axis=1, keepdims=True))
    hi2, lo2 = _halves(l2)
    cand += _top_candidates(hi2, 3)
    cand.append(jnp.max(lo2, axis=1, keepdims=True))
    hi3, lo3 = _halves(l3)
    cand += _top_candidates(hi3, 2)
    cand.append(jnp.max(lo3, axis=1, keepdims=True))
    cand.append(jnp.max(l4, axis=1, keepdims=True))

    candv = jnp.concatenate(cand, axis=1)
    num = jnp.zeros((BM, 1), jnp.float32)
    for it in range(5):
        m = jnp.max(candv, axis=1, keepdims=True)
        num = num + m.astype(jnp.float32)
        if it + 1 < 5:
            candv = jnp.where(candv == m, jnp.bfloat16(0.0), candv)

    loss = jnp.log(den / num)
    out_ref[...] += jnp.sum(loss, axis=0, keepdims=True).reshape(1, 1)


@jax.jit
def kernel(z):
    out = pl.pallas_call(
        _and_kernel,
        grid=(GRID,),
        in_specs=[pl.BlockSpec((N, D), lambda i: (0, 0))],
        out_specs=pl.BlockSpec((1, 1), lambda i: (0, 0)),
        out_shape=jax.ShapeDtypeStruct((1, 1), jnp.float32),
        scratch_shapes=[pltpu.VMEM((N, D), jnp.bfloat16)],
    )(z)
    return out[0, 0] * (1.0 / N)


# speedup vs baseline: 1.0075x; 1.0075x over previous
import jax
import jax.numpy as jnp
from jax.experimental import pallas as pl
from jax.experimental.pallas import tpu as pltpu

TEMP = 0.1
N = 4096
D = 256
BM = 2048
GRID = N // BM
EXP2_SCALE = 1.0 / (TEMP * 0.6931471805599453)
SQRT_SCALE = EXP2_SCALE ** 0.5


def _top_candidates(arr, n_top):
    cand = []
    cur = arr
    zero = jnp.array(0.0, arr.dtype)
    for it in range(n_top):
        m = jnp.max(cur, axis=1, keepdims=True)
        cand.append(m)
        if it + 1 < n_top:
            cur = jnp.where(cur == m, zero, cur)
    return cand


def _sort4(v1, v2, v3, v4):
    s1, t1 = jnp.maximum(v1, v2), jnp.minimum(v1, v2)
    s2, t2 = jnp.maximum(v3, v4), jnp.minimum(v3, v4)
    l1, x = jnp.maximum(s1, s2), jnp.minimum(s1, s2)
    y, l4 = jnp.maximum(t1, t2), jnp.minimum(t1, t2)
    l2, l3 = jnp.maximum(x, y), jnp.minimum(x, y)
    return l1, l2, l3, l4


def _quarters(arr):
    q = arr.shape[1] // 4
    return arr[:, 0:q], arr[:, q:2 * q], arr[:, 2 * q:3 * q], arr[:, 3 * q:]


def _halves(arr):
    h = arr.shape[1] // 2
    a, b = arr[:, 0:h], arr[:, h:]
    return jnp.maximum(a, b), jnp.minimum(a, b)


def _and_kernel(z_ref, out_ref, zn_ref):
    i = pl.program_id(0)

    @pl.when(i == 0)
    def _init():
        zf = z_ref[...]
        nrm = jnp.sqrt(jnp.sum(zf * zf, axis=1, keepdims=True))
        zn = zf / jnp.maximum(nrm, 1e-12)
        zn_ref[...] = (zn * SQRT_SCALE).astype(jnp.bfloat16)
        out_ref[...] = jnp.zeros_like(out_ref)

    zn_blk = zn_ref[pl.ds(i * BM, BM), :]
    sim = jax.lax.dot_general(
        zn_blk,
        zn_ref[...],
        dimension_numbers=(((1,), (1,)), ((), ())),
        preferred_element_type=jnp.float32,
    )

    m0 = jnp.max(sim, axis=1, keepdims=True)
    e = jnp.exp2(sim - m0)

    eb = e.astype(jnp.bfloat16)
    ones = jnp.ones((N, 1), jnp.bfloat16)
    den = jax.lax.dot_general(
        eb,
        ones,
        dimension_numbers=(((1,), (0,)), ((), ())),
        preferred_element_type=jnp.float32,
    ) - 1.0
    l1, l2, l3, l4 = _sort4(*_quarters(eb))
    b1, b2, b3, b4 = _sort4(*_quarters(l1))

    cand = []
    one = jnp.array(1.0, b1.dtype)
    zero = jnp.array(0.0, b1.dtype)
    cand += _top_candidates(jnp.where(b1 == one, zero, b1), 5)
    cand += _top_candidates(b2, 3)
    cand += _top_candidates(b3, 2)
    cand.append(jnp.max(b4, axis=1, keepdims=True))
    hi2, lo2 = _halves(l2)
    cand += _top_candidates(hi2, 3)
    cand.append(jnp.max(lo2, axis=1, keepdims=True))
    hi3, lo3 = _halves(l3)
    cand += _top_candidates(hi3, 2)
    cand.append(jnp.max(lo3, axis=1, keepdims=True))
    cand.append(jnp.max(l4, axis=1, keepdims=True))

    candv = jnp.concatenate(cand, axis=1)
    num = jnp.zeros((BM, 1), jnp.float32)
    for it in range(5):
        m = jnp.max(candv, axis=1, keepdims=True)
        num = num + m.astype(jnp.float32)
        if it + 1 < 5:
            candv = jnp.where(candv == m, jnp.bfloat16(0.0), candv)

    loss = jnp.log(den / num)
    out_ref[...] += jnp.sum(loss, axis=0, keepdims=True).reshape(1, 1)


@jax.jit
def kernel(z):
    out = pl.pallas_call(
        _and_kernel,
        grid=(GRID,),
        in_specs=[pl.BlockSpec((N, D), lambda i: (0, 0))],
        out_specs=pl.BlockSpec((1, 1), lambda i: (0, 0)),
        out_shape=jax.ShapeDtypeStruct((1, 1), jnp.float32),
        scratch_shapes=[pltpu.VMEM((N, D), jnp.bfloat16)],
    )(z)
    return out[0, 0] * (1.0 / N)


# speedup vs baseline: 1.0183x; 1.0108x over previous
import jax
import jax.numpy as jnp
from jax.experimental import pallas as pl
from jax.experimental.pallas import tpu as pltpu

TEMP = 0.1
N = 4096
D = 256
BM = 2048
GRID = N // BM
EXP2_SCALE = 1.0 / (TEMP * 0.6931471805599453)
SQRT_SCALE = EXP2_SCALE ** 0.5


def _top_candidates(arr, n_top):
    cand = []
    cur = arr
    zero = jnp.array(0.0, arr.dtype)
    for it in range(n_top):
        m = jnp.max(cur, axis=1, keepdims=True)
        cand.append(m)
        if it + 1 < n_top:
            cur = jnp.where(cur == m, zero, cur)
    return cand


def _sort4(v1, v2, v3, v4):
    s1, t1 = jnp.maximum(v1, v2), jnp.minimum(v1, v2)
    s2, t2 = jnp.maximum(v3, v4), jnp.minimum(v3, v4)
    l1, x = jnp.maximum(s1, s2), jnp.minimum(s1, s2)
    y, l4 = jnp.maximum(t1, t2), jnp.minimum(t1, t2)
    l2, l3 = jnp.maximum(x, y), jnp.minimum(x, y)
    return l1, l2, l3, l4


def _quarters(arr):
    q = arr.shape[1] // 4
    return arr[:, 0:q], arr[:, q:2 * q], arr[:, 2 * q:3 * q], arr[:, 3 * q:]


def _halves(arr):
    h = arr.shape[1] // 2
    a, b = arr[:, 0:h], arr[:, h:]
    return jnp.maximum(a, b), jnp.minimum(a, b)


def _and_kernel(z_ref, out_ref, zn_ref):
    i = pl.program_id(0)

    @pl.when(i == 0)
    def _init():
        zf = z_ref[...]
        nrm = jnp.sqrt(jnp.sum(zf * zf, axis=1, keepdims=True))
        zn = zf / jnp.maximum(nrm, 1e-12)
        zn_ref[...] = (zn * SQRT_SCALE).astype(jnp.bfloat16)
        out_ref[...] = jnp.zeros_like(out_ref)

    zn_blk = zn_ref[pl.ds(i * BM, BM), :]
    sim = jax.lax.dot_general(
        zn_blk,
        zn_ref[...],
        dimension_numbers=(((1,), (1,)), ((), ())),
        preferred_element_type=jnp.float32,
    )

    m0 = jnp.max(sim, axis=1, keepdims=True)
    e = jnp.exp2(sim - m0)

    eb = e.astype(jnp.bfloat16)
    l1, l2, l3, l4 = _sort4(*_quarters(eb))
    one = jnp.array(1.0, l1.dtype)
    zero = jnp.array(0.0, l1.dtype)
    l1z = jnp.where(l1 == one, zero, l1)

    lsum = (l1z + l2) + (l3 + l4)
    ones = jnp.ones((N // 4, 1), jnp.bfloat16)
    den = jax.lax.dot_general(
        lsum,
        ones,
        dimension_numbers=(((1,), (0,)), ((), ())),
        preferred_element_type=jnp.float32,
    )

    b1, b2, b3, b4 = _sort4(*_quarters(l1z))

    cand = []
    cand += _top_candidates(b1, 5)
    cand += _top_candidates(b2, 2)
    cand.append(jnp.max(b3, axis=1, keepdims=True))
    cand.append(jnp.max(b4, axis=1, keepdims=True))
    hi2, lo2 = _halves(l2)
    cand += _top_candidates(hi2, 3)
    cand.append(jnp.max(lo2, axis=1, keepdims=True))
    hi3, lo3 = _halves(l3)
    cand += _top_candidates(hi3, 2)
    cand.append(jnp.max(lo3, axis=1, keepdims=True))
    cand.append(jnp.max(l4, axis=1, keepdims=True))

    candv = jnp.concatenate(cand, axis=1)
    num = jnp.zeros((BM, 1), jnp.float32)
    for it in range(5):
        m = jnp.max(candv, axis=1, keepdims=True)
        num = num + m.astype(jnp.float32)
        if it + 1 < 5:
            candv = jnp.where(candv == m, jnp.bfloat16(0.0), candv)

    loss = jnp.log(den / num)
    out_ref[...] += jnp.sum(loss, axis=0, keepdims=True).reshape(1, 1)


@jax.jit
def kernel(z):
    out = pl.pallas_call(
        _and_kernel,
        grid=(GRID,),
        in_specs=[pl.BlockSpec((N, D), lambda i: (0, 0))],
        out_specs=pl.BlockSpec((1, 1), lambda i: (0, 0)),
        out_shape=jax.ShapeDtypeStruct((1, 1), jnp.float32),
        scratch_shapes=[pltpu.VMEM((N, D), jnp.bfloat16)],
    )(z)
    return out[0, 0] * (1.0 / N)
